# baseline (speedup 1.0000x reference)
import functools

import jax
import jax.numpy as jnp
from jax import lax
from jax.experimental import pallas as pl
from jax.experimental.pallas import tpu as pltpu
from jax.experimental.pallas import tpu_sc as plsc

_S = 32
_DT = 256
_QA = 128
_PT = 1024
_QT = 128
_CHUNK = 128


def _topk_body(n2, s, x2_ref, x1_ref, ind_ref):
    b = pl.program_id(0)
    x2 = x2_ref[0]
    x1 = x1_ref[0]
    qa = x1.shape[1]
    xx = (x2[0] * x2[0] + x2[1] * x2[1]) + x2[2] * x2[2]
    yy = (x1[0] * x1[0] + x1[1] * x1[1]) + x1[2] * x1[2]
    x2b = x2[0:3].astype(jnp.bfloat16).astype(jnp.float32)
    x1b = x1[0:3].astype(jnp.bfloat16).astype(jnp.float32)
    p0 = x2b[0][:, None] * x1b[0][None, :]
    p1 = x2b[1][:, None] * x1b[1][None, :]
    p2 = x2b[2][:, None] * x1b[2][None, :]
    mmax = jnp.maximum(jnp.maximum(jnp.abs(p0), jnp.abs(p1)), jnp.abs(p2))
    mmax = jnp.maximum(mmax, jnp.float32(2.0 ** -90))
    ub = lax.bitcast_convert_type(mmax, jnp.int32) & jnp.int32(0x7F800000)
    bits_q = ub - jnp.int32(27 << 23)
    bits_qinv = jnp.int32(0x7F000000) - bits_q
    q = lax.bitcast_convert_type(bits_q, jnp.float32)
    qinv = lax.bitcast_convert_type(bits_qinv, jnp.float32)
    t = (jnp.trunc(p0 * qinv).astype(jnp.int32)
         + jnp.trunc(p1 * qinv).astype(jnp.int32)
         + jnp.trunc(p2 * qinv).astype(jnp.int32))
    dot = t.astype(jnp.float32) * q
    d = (xx[:, None] + yy[None, :]) - 2.0 * dot
    d = jnp.maximum(d, 0.0)
    iota0 = lax.broadcasted_iota(jnp.int32, (n2, qa), 0)
    srow = lax.broadcasted_iota(jnp.int32, (s, qa), 0)

    def body(j, carry):
        dd, ind = carry
        m = jnp.min(dd, axis=0, keepdims=True)
        cand = jnp.where(dd == m, iota0, n2)
        idx = jnp.min(cand, axis=0, keepdims=True)
        ind = jnp.where(srow == j, idx, ind)
        dd = jnp.where(iota0 == idx, jnp.float32(jnp.inf), dd)
        return dd, ind

    _, ind = lax.fori_loop(0, s, body, (d, jnp.zeros((s, qa), jnp.int32)))
    ind_ref[0] = ind.T + b * n2


def _topk_call(x2p, x1p, n, s):
    b = x2p.shape[0]
    grid = (b, n // _QA)
    return pl.pallas_call(
        functools.partial(_topk_body, n, s),
        grid=grid,
        in_specs=[
            pl.BlockSpec((1, 8, n), lambda bi, qi: (bi, 0, 0)),
            pl.BlockSpec((1, 8, _QA), lambda bi, qi: (bi, 0, qi)),
        ],
        out_specs=pl.BlockSpec((1, _QA, s), lambda bi, qi: (bi, qi, 0)),
        out_shape=jax.ShapeDtypeStruct((b, n, s), jnp.int32),
    )(x2p, x1p)


def _tab_body(x2_ref, f_ref, tab_ref):
    x2t = x2_ref[0].T
    ft = f_ref[0].T
    pt = x2t.shape[0]
    pad = jnp.zeros((pt, _DT - 3 - ft.shape[1]), jnp.float32)
    tab_ref[0] = jnp.concatenate([x2t[:, 0:3], ft, pad], axis=1)


def _tab_call(x2p, features, n, c):
    b = x2p.shape[0]
    grid = (b, n // _PT)
    return pl.pallas_call(
        _tab_body,
        grid=grid,
        in_specs=[
            pl.BlockSpec((1, 8, _PT), lambda bi, pi: (bi, 0, pi)),
            pl.BlockSpec((1, c, _PT), lambda bi, pi: (bi, 0, pi)),
        ],
        out_specs=pl.BlockSpec((1, _PT, _DT), lambda bi, pi: (bi, pi, 0)),
        out_shape=jax.ShapeDtypeStruct((b, n, _DT), jnp.float32),
    )(x2p, features)


def _sc_gather_call(tab, idx_flat):
    tot = idx_flat.shape[0]
    nw = 32
    per_w = tot // nw
    n_steps = per_w // _CHUNK
    mesh = plsc.VectorSubcoreMesh(core_axis_name="c", subcore_axis_name="s")

    @functools.partial(
        pl.kernel,
        out_type=jax.ShapeDtypeStruct((tot, _DT), jnp.float32),
        mesh=mesh,
        scratch_types=[
            pltpu.VMEM((_CHUNK,), jnp.int32),
            pltpu.VMEM((_CHUNK, _DT), jnp.float32),
            pltpu.SemaphoreType.DMA,
        ],
    )
    def body(tab_hbm, idx_hbm, out_hbm, idx_v, rows_v, sem):
        wid = lax.axis_index("s") * 2 + lax.axis_index("c")

        def step(i, carry):
            base = pl.multiple_of(wid * per_w + i * _CHUNK, _CHUNK)
            pltpu.sync_copy(idx_hbm.at[pl.ds(base, _CHUNK)], idx_v)
            pltpu.async_copy(tab_hbm.at[idx_v], rows_v, sem).wait()
            pltpu.sync_copy(rows_v, out_hbm.at[pl.ds(base, _CHUNK)])
            return carry

        lax.fori_loop(0, n_steps, step, 0)

    return body(tab, idx_flat)


def _regroup_body(co, s, g_ref, x1_ref, out_ref):
    g = g_ref[0][:, 0:144]
    x1t = x1_ref[0].T
    qt = x1t.shape[0]
    xr = jnp.broadcast_to(x1t[:, None, 0:3], (qt, s, 3)).reshape(qt * s, 3)
    top = g[:, 0:3] - xr
    full = jnp.concatenate([top, g[:, 3:]], axis=1)
    out_ref[0] = full.T[0:co]


def _regroup_call(g3, x1p, n, c, s):
    b = x1p.shape[0]
    co = c + 3
    grid = (b, n // _QT)
    return pl.pallas_call(
        functools.partial(_regroup_body, co, s),
        grid=grid,
        in_specs=[
            pl.BlockSpec((1, _QT * s, _DT), lambda bi, qi: (bi, qi, 0)),
            pl.BlockSpec((1, 8, _QT), lambda bi, qi: (bi, 0, qi)),
        ],
        out_specs=pl.BlockSpec((1, co, _QT * s), lambda bi, qi: (bi, 0, qi)),
        out_shape=jax.ShapeDtypeStruct((b, co, n * s), jnp.float32),
    )(g3, x1p)


def kernel(xyz2, xyz1, features):
    b, _, n = xyz2.shape
    c = features.shape[1]
    s = _S
    zpad = jnp.zeros((b, 5, n), jnp.float32)
    x2p = jnp.concatenate([xyz2, zpad], axis=1)
    x1p = jnp.concatenate([xyz1, zpad], axis=1)

    gidx = _topk_call(x2p, x1p, n, s)
    tab = _tab_call(x2p, features, n, c).reshape(b * n, _DT)
    g = _sc_gather_call(tab, gidx.reshape(b * n * s))
    out = _regroup_call(g.reshape(b, n * s, _DT), x1p, n, c, s)
    return out.reshape(b, c + 3, n, s)


# speedup vs baseline: 1.2651x; 1.2651x over previous
import functools

import jax
import jax.numpy as jnp
from jax import lax
from jax.experimental import pallas as pl
from jax.experimental.pallas import tpu as pltpu
from jax.experimental.pallas import tpu_sc as plsc

_S = 32
_DT = 256
_QA = 256
_PT = 1024
_QT = 128
_CHUNK = 128


def _topk_body(n2, s, x2_ref, x1_ref, ind_ref):
    b = pl.program_id(0)
    x2 = x2_ref[0]
    x1 = x1_ref[0]
    qa = x1.shape[1]
    xx = (x2[0] * x2[0] + x2[1] * x2[1]) + x2[2] * x2[2]
    yy = (x1[0] * x1[0] + x1[1] * x1[1]) + x1[2] * x1[2]
    x2b = x2[0:3].astype(jnp.bfloat16).astype(jnp.float32)
    x1b = x1[0:3].astype(jnp.bfloat16).astype(jnp.float32)
    p0 = x2b[0][:, None] * x1b[0][None, :]
    p1 = x2b[1][:, None] * x1b[1][None, :]
    p2 = x2b[2][:, None] * x1b[2][None, :]
    mmax = jnp.maximum(jnp.maximum(jnp.abs(p0), jnp.abs(p1)), jnp.abs(p2))
    mmax = jnp.maximum(mmax, jnp.float32(2.0 ** -90))
    ub = lax.bitcast_convert_type(mmax, jnp.int32) & jnp.int32(0x7F800000)
    bits_q = ub - jnp.int32(27 << 23)
    bits_qinv = jnp.int32(0x7F000000) - bits_q
    q = lax.bitcast_convert_type(bits_q, jnp.float32)
    qinv = lax.bitcast_convert_type(bits_qinv, jnp.float32)
    t = (jnp.trunc(p0 * qinv).astype(jnp.int32)
         + jnp.trunc(p1 * qinv).astype(jnp.int32)
         + jnp.trunc(p2 * qinv).astype(jnp.int32))
    dot = t.astype(jnp.float32) * q
    d = (xx[:, None] + yy[None, :]) - 2.0 * dot
    d = jnp.maximum(d, 0.0)
    iota0 = lax.broadcasted_iota(jnp.int32, (n2, qa), 0)
    srow = lax.broadcasted_iota(jnp.int32, (s, qa), 0)

    def body(j, carry):
        dd, ind = carry
        m = jnp.min(dd, axis=0, keepdims=True)
        cand = jnp.where(dd == m, iota0, n2)
        idx = jnp.min(cand, axis=0, keepdims=True)
        ind = jnp.where(srow == j, idx, ind)
        dd = jnp.where(iota0 == idx, jnp.float32(jnp.inf), dd)
        return dd, ind

    _, ind = lax.fori_loop(0, s, body, (d, jnp.zeros((s, qa), jnp.int32)))
    ind_ref[0] = ind.T + b * n2


def _topk_call(x2p, x1p, n, s):
    b = x2p.shape[0]
    grid = (b, n // _QA)
    return pl.pallas_call(
        functools.partial(_topk_body, n, s),
        grid=grid,
        in_specs=[
            pl.BlockSpec((1, 8, n), lambda bi, qi: (bi, 0, 0)),
            pl.BlockSpec((1, 8, _QA), lambda bi, qi: (bi, 0, qi)),
        ],
        out_specs=pl.BlockSpec((1, _QA, s), lambda bi, qi: (bi, qi, 0)),
        out_shape=jax.ShapeDtypeStruct((b, n, s), jnp.int32),
    )(x2p, x1p)


def _tab_body(x2_ref, f_ref, tab_ref):
    x2t = x2_ref[0].T
    ft = f_ref[0].T
    pt = x2t.shape[0]
    pad = jnp.zeros((pt, _DT - 3 - ft.shape[1]), jnp.float32)
    tab_ref[0] = jnp.concatenate([x2t[:, 0:3], ft, pad], axis=1)


def _tab_call(x2p, features, n, c):
    b = x2p.shape[0]
    grid = (b, n // _PT)
    return pl.pallas_call(
        _tab_body,
        grid=grid,
        in_specs=[
            pl.BlockSpec((1, 8, _PT), lambda bi, pi: (bi, 0, pi)),
            pl.BlockSpec((1, c, _PT), lambda bi, pi: (bi, 0, pi)),
        ],
        out_specs=pl.BlockSpec((1, _PT, _DT), lambda bi, pi: (bi, pi, 0)),
        out_shape=jax.ShapeDtypeStruct((b, n, _DT), jnp.float32),
    )(x2p, features)


def _sc_gather_call(tab, idx_flat):
    tot = idx_flat.shape[0]
    nw = 32
    per_w = tot // nw
    n_steps = per_w // _CHUNK
    mesh = plsc.VectorSubcoreMesh(core_axis_name="c", subcore_axis_name="s")

    @functools.partial(
        pl.kernel,
        out_type=jax.ShapeDtypeStruct((tot, _DT), jnp.float32),
        mesh=mesh,
        scratch_types=[
            pltpu.VMEM((_CHUNK,), jnp.int32),
            pltpu.VMEM((_CHUNK, _DT), jnp.float32),
            pltpu.SemaphoreType.DMA,
        ],
    )
    def body(tab_hbm, idx_hbm, out_hbm, idx_v, rows_v, sem):
        wid = lax.axis_index("s") * 2 + lax.axis_index("c")

        def step(i, carry):
            base = pl.multiple_of(wid * per_w + i * _CHUNK, _CHUNK)
            pltpu.sync_copy(idx_hbm.at[pl.ds(base, _CHUNK)], idx_v)
            pltpu.async_copy(tab_hbm.at[idx_v], rows_v, sem).wait()
            pltpu.sync_copy(rows_v, out_hbm.at[pl.ds(base, _CHUNK)])
            return carry

        lax.fori_loop(0, n_steps, step, 0)

    return body(tab, idx_flat)


def _regroup_body(co, s, g_ref, x1_ref, out_ref):
    g = g_ref[0][:, 0:144]
    x1t = x1_ref[0].T
    qt = x1t.shape[0]
    xr = jnp.broadcast_to(x1t[:, None, 0:3], (qt, s, 3)).reshape(qt * s, 3)
    top = g[:, 0:3] - xr
    full = jnp.concatenate([top, g[:, 3:]], axis=1)
    out_ref[0] = full.T[0:co]


def _regroup_call(g3, x1p, n, c, s):
    b = x1p.shape[0]
    co = c + 3
    grid = (b, n // _QT)
    return pl.pallas_call(
        functools.partial(_regroup_body, co, s),
        grid=grid,
        in_specs=[
            pl.BlockSpec((1, _QT * s, _DT), lambda bi, qi: (bi, qi, 0)),
            pl.BlockSpec((1, 8, _QT), lambda bi, qi: (bi, 0, qi)),
        ],
        out_specs=pl.BlockSpec((1, co, _QT * s), lambda bi, qi: (bi, 0, qi)),
        out_shape=jax.ShapeDtypeStruct((b, co, n * s), jnp.float32),
    )(g3, x1p)


def kernel(xyz2, xyz1, features):
    b, _, n = xyz2.shape
    c = features.shape[1]
    s = _S
    zpad = jnp.zeros((b, 5, n), jnp.float32)
    x2p = jnp.concatenate([xyz2, zpad], axis=1)
    x1p = jnp.concatenate([xyz1, zpad], axis=1)

    gidx = _topk_call(x2p, x1p, n, s)
    tab = _tab_call(x2p, features, n, c).reshape(b * n, _DT)
    g = _sc_gather_call(tab, gidx.reshape(b * n * s))
    out = _regroup_call(g.reshape(b, n * s, _DT), x1p, n, c, s)
    return out.reshape(b, c + 3, n, s)
